# Initial kernel scaffold; baseline (speedup 1.0000x reference)
import jax
import jax.numpy as jnp
from jax.experimental import pallas as pl


def kernel(x, edge_index, W0, b0, g0, be0, W1, b1, g1, be1, W2, b2, g2, be2, Wc1, bc1, Wc2, bc2):
    raise NotImplementedError("write your pallas kernel here")


# speedup vs baseline: 8.6171x; 8.6171x over previous
import functools

import jax
import jax.numpy as jnp
from jax import lax
from jax.experimental import pallas as pl
from jax.experimental.pallas import tpu as pltpu
from jax.experimental.pallas import tpu_sc as plsc

N = 10000
E = 320000
H = 128
C = 2

NC = 2
NS = 16
NW = NC * NS
K = 128
G = -(-(E // NW) // K)
EPAD = NW * G * K
NPAD = 10240
RB = 1024
GRID = NPAD // RB
RPW = NPAD // NS
EPS = 1e-5

_mesh = plsc.VectorSubcoreMesh(core_axis_name="c", subcore_axis_name="s")


@functools.partial(
    pl.kernel,
    out_type=jax.ShapeDtypeStruct((NC, NPAD, 1), jnp.float32),
    mesh=_mesh,
    scratch_types=[
        pltpu.VMEM((K,), jnp.int32),
        pltpu.VMEM((K, 1), jnp.float32),
        pltpu.VMEM_SHARED((NPAD, 1), jnp.float32),
    ],
)
def _sc_deg(dsts, ones_hbm, zeros_hbm, deg_out, idx_v, ones_v, acc):
    c = lax.axis_index("c")
    s = lax.axis_index("s")
    wid = c * NS + s
    r0 = s * RPW
    pltpu.sync_copy(ones_hbm, ones_v)
    pltpu.sync_copy(zeros_hbm, acc.at[pl.ds(r0, RPW), :])
    plsc.subcore_barrier()

    def body(g, carry):
        pltpu.sync_copy(dsts.at[wid, g], idx_v)
        pltpu.sync_copy(ones_v, acc.at[idx_v], add=True)
        return carry

    lax.fori_loop(0, G, body, 0)
    plsc.subcore_barrier()
    pltpu.sync_copy(acc.at[pl.ds(r0, RPW), :], deg_out.at[c, pl.ds(r0, RPW), :])


@functools.partial(
    pl.kernel,
    out_type=jax.ShapeDtypeStruct((NC, NPAD, H), jnp.float32),
    mesh=_mesh,
    scratch_types=[
        pltpu.VMEM((K,), jnp.int32),
        pltpu.VMEM((K,), jnp.int32),
        pltpu.VMEM((K, H), jnp.float32),
        pltpu.SemaphoreType.DMA,
        pltpu.VMEM_SHARED((NPAD, H), jnp.float32),
    ],
)
def _sc_scatter(q, srcs, dsts, out, sidx_v, didx_v, rows_v, sem, acc):
    c = lax.axis_index("c")
    s = lax.axis_index("s")
    wid = c * NS + s
    r0 = s * RPW
    pltpu.sync_copy(q.at[pl.ds(r0, RPW), :], acc.at[pl.ds(r0, RPW), :])
    plsc.subcore_barrier()

    def body(g, carry):
        pltpu.sync_copy(srcs.at[wid, g], sidx_v)
        pltpu.sync_copy(dsts.at[wid, g], didx_v)
        pltpu.async_copy(q.at[sidx_v], rows_v, sem).wait()
        pltpu.sync_copy(rows_v, acc.at[didx_v], add=True)
        return carry

    lax.fori_loop(0, G, body, 0)
    plsc.subcore_barrier()
    pltpu.sync_copy(acc.at[pl.ds(r0, RPW), :], out.at[c, pl.ds(r0, RPW), :])


def _dinv_block(degp):
    return lax.rsqrt(1.0 + degp[0] + degp[1])


def _tc0_body(x_ref, w_ref, degp_ref, o_ref):
    dinv = _dinv_block(degp_ref[...])
    o_ref[...] = dinv * jnp.dot(x_ref[...], w_ref[...],
                                preferred_element_type=jnp.float32)


def _ln_relu(p, q, degp, b, g, be):
    dinv = _dinv_block(degp)
    t = dinv * (p[0] + p[1] - q) + b
    m = jnp.mean(t, axis=-1, keepdims=True)
    v = jnp.mean((t - m) ** 2, axis=-1, keepdims=True)
    y = (t - m) * lax.rsqrt(v + EPS) * g + be
    return jnp.maximum(y, 0.0)


def _epi_body(p_ref, q_ref, degp_ref, b_ref, g_ref, be_ref, w_ref, o_ref):
    h = _ln_relu(p_ref[...], q_ref[...], degp_ref[...],
                 b_ref[...], g_ref[...], be_ref[...])
    dinv = _dinv_block(degp_ref[...])
    o_ref[...] = dinv * jnp.dot(h, w_ref[...],
                                preferred_element_type=jnp.float32)


def _fin_body(p_ref, q_ref, degp_ref, b_ref, g_ref, be_ref,
              wc1_ref, bc1_ref, wc2_ref, bc2_ref, o_ref):
    h = _ln_relu(p_ref[...], q_ref[...], degp_ref[...],
                 b_ref[...], g_ref[...], be_ref[...])
    z = jnp.maximum(jnp.dot(h, wc1_ref[...],
                            preferred_element_type=jnp.float32) + bc1_ref[...],
                    0.0)
    o_ref[...] = jnp.dot(z, wc2_ref[...],
                         preferred_element_type=jnp.float32) + bc2_ref[...]


_rows = pl.BlockSpec((RB, H), lambda i: (i, 0))
_degp_spec = pl.BlockSpec((NC, RB, 1), lambda i: (0, i, 0))
_p_spec = pl.BlockSpec((NC, RB, H), lambda i: (0, i, 0))


def _full(shape):
    return pl.BlockSpec(shape, lambda i: tuple(0 for _ in shape))


def _tc0(xp, W, degp):
    return pl.pallas_call(
        _tc0_body,
        grid=(GRID,),
        in_specs=[_rows, _full((H, H)), _degp_spec],
        out_specs=_rows,
        out_shape=jax.ShapeDtypeStruct((NPAD, H), jnp.float32),
    )(xp, W, degp)


def _tc_epi(p, q, degp, b, g, be, Wn):
    return pl.pallas_call(
        _epi_body,
        grid=(GRID,),
        in_specs=[_p_spec, _rows, _degp_spec,
                  _full((1, H)), _full((1, H)), _full((1, H)), _full((H, H))],
        out_specs=_rows,
        out_shape=jax.ShapeDtypeStruct((NPAD, H), jnp.float32),
    )(p, q, degp, b, g, be, Wn)


def _tc_fin(p, q, degp, b, g, be, Wc1, bc1, Wc2p, bc2p):
    return pl.pallas_call(
        _fin_body,
        grid=(GRID,),
        in_specs=[_p_spec, _rows, _degp_spec,
                  _full((1, H)), _full((1, H)), _full((1, H)),
                  _full((H, H // 2)), _full((1, H // 2)),
                  _full((H // 2, H)), _full((1, H))],
        out_specs=_rows,
        out_shape=jax.ShapeDtypeStruct((NPAD, H), jnp.float32),
    )(p, q, degp, b, g, be, Wc1, bc1, Wc2p, bc2p)


def kernel(x, edge_index, W0, b0, g0, be0, W1, b1, g1, be1,
           W2, b2, g2, be2, Wc1, bc1, Wc2, bc2):
    f32 = jnp.float32
    pad = EPAD - E
    srcs = jnp.concatenate(
        [edge_index[0], jnp.zeros((pad,), jnp.int32)]).reshape(NW, G, K)
    dsts = jnp.concatenate(
        [edge_index[1], jnp.full((pad,), N, jnp.int32)]).reshape(NW, G, K)
    xp = jnp.pad(x, ((0, NPAD - N), (0, 0)))
    ones_col = jnp.ones((K, 1), f32)
    zeros_col = jnp.zeros((RPW, 1), f32)

    degp = _sc_deg(dsts, ones_col, zeros_col)

    q0 = _tc0(xp, W0, degp)
    p0 = _sc_scatter(q0, srcs, dsts)
    q1 = _tc_epi(p0, q0, degp, b0.reshape(1, H), g0.reshape(1, H),
                 be0.reshape(1, H), W1)
    p1 = _sc_scatter(q1, srcs, dsts)
    q2 = _tc_epi(p1, q1, degp, b1.reshape(1, H), g1.reshape(1, H),
                 be1.reshape(1, H), W2)
    p2 = _sc_scatter(q2, srcs, dsts)

    Wc2p = jnp.pad(Wc2, ((0, 0), (0, H - C)))
    bc2p = jnp.pad(bc2, (0, H - C)).reshape(1, H)
    out = _tc_fin(p2, q2, degp, b2.reshape(1, H), g2.reshape(1, H),
                  be2.reshape(1, H), Wc1, bc1.reshape(1, H // 2), Wc2p, bc2p)
    return out[:N, :C]
